# Initial kernel scaffold; baseline (speedup 1.0000x reference)
import functools

import numpy as np
import jax
import jax.numpy as jnp
from jax import lax
from jax.experimental import pallas as pl
from jax.experimental.pallas import tpu as pltpu
from jax.experimental.pallas import tpu_sc as plsc

N_NODES = 10000
N_EDGES = 160000
NS, NV = 16, 4
R_EMB_DIM, R_EMB_MAX = 32, 10.0
OUT_CH = 32

EPAD = 163840
NW = 32
PER_W = EPAD // NW
CH = 128
EB = 1280
NBLK = EPAD // EB
NREAL = N_EDGES // EB
NB = 2000

F32 = jnp.float32
_SQ3 = float(np.sqrt(3.0))
_A0 = 1.0 / float(np.sqrt(NS + NV))
_A1 = 1.0 / float(np.sqrt(NS + 2 * NV))

_S416 = np.ones((416,), np.float32)
_S416[0:256] = _A0
_S416[256:320] = _A0 / _SQ3
_S416[320:384] = _A1
_S416[384:400] = _A1
_S416[400:416] = _A1 / float(np.sqrt(2.0))

_IDXS = np.concatenate([np.arange(OUT_CH), np.repeat(np.arange(OUT_CH), 3)])
_ORDR = np.concatenate([np.zeros(OUT_CH, np.int64), np.tile(np.arange(1, 4), OUT_CH)])
_OUT_PERM = np.argsort(_IDXS * 4 + _ORDR, kind="stable")

_RBF_OFFS = np.linspace(0.0, R_EMB_MAX, R_EMB_DIM).astype(np.float32)
_RBF_COEFF = float(-0.5 / (_RBF_OFFS[1] - _RBF_OFFS[0]) ** 2)


def _tt(x):
    eye = jnp.eye(x.shape[1], dtype=F32)
    return lax.dot_general(eye, x, (((1,), (1,)), ((), ())),
                           preferred_element_type=F32)


def _mm(a, b):
    return jnp.dot(a, b, preferred_element_type=F32)


def _sc_mesh():
    return plsc.VectorSubcoreMesh(core_axis_name="c", subcore_axis_name="s")


def _sc_gather_pair(tab_a, idx_a, tab_b, idx_b):
    wa, wb = tab_a.shape[1], tab_b.shape[1]

    @functools.partial(
        pl.kernel,
        mesh=_sc_mesh(),
        out_type=(jax.ShapeDtypeStruct((EPAD, wa), F32),
                  jax.ShapeDtypeStruct((EPAD, wb), F32)),
        scratch_types=[pltpu.VMEM((CH,), jnp.int32),
                       pltpu.VMEM((CH,), jnp.int32),
                       pltpu.VMEM((CH, wa), F32),
                       pltpu.VMEM((CH, wb), F32),
                       pltpu.SemaphoreType.DMA,
                       pltpu.SemaphoreType.DMA],
    )
    def k(ta, ia_h, tb, ib_h, oa, ob, ia_v, ib_v, ra, rb, s1, s2):
        wid = lax.axis_index("s") * 2 + lax.axis_index("c")
        base = wid * PER_W

        @pl.loop(0, PER_W, step=CH)
        def _(off):
            pltpu.sync_copy(ia_h.at[pl.ds(base + off, CH)], ia_v)
            pltpu.sync_copy(ib_h.at[pl.ds(base + off, CH)], ib_v)
            c1 = pltpu.async_copy(ta.at[ia_v], ra, s1)
            c2 = pltpu.async_copy(tb.at[ib_v], rb, s2)
            c1.wait()
            c2.wait()
            pltpu.sync_copy(ra, oa.at[pl.ds(base + off, CH)])
            pltpu.sync_copy(rb, ob.at[pl.ds(base + off, CH)])

    return k(tab_a, idx_a, tab_b, idx_b)


def _sc_scatter_add(tp, idx, zeros_tab):
    rps = N_NODES // 16

    @functools.partial(
        pl.kernel,
        mesh=_sc_mesh(),
        out_type=jax.ShapeDtypeStruct((2, N_NODES, 32), F32),
        scratch_types=[pltpu.VMEM((CH,), jnp.int32),
                       pltpu.VMEM((CH, 32), F32),
                       pltpu.VMEM_SHARED((N_NODES, 32), F32)],
    )
    def k(tp_h, idx_h, z_h, out_h, idx_v, rows_v, shared):
        ci = lax.axis_index("c")
        si = lax.axis_index("s")
        wid = si * 2 + ci
        pltpu.sync_copy(z_h.at[pl.ds(si * rps, rps)],
                        shared.at[pl.ds(si * rps, rps)])
        plsc.subcore_barrier()
        base = wid * PER_W

        @pl.loop(0, PER_W, step=CH)
        def _(off):
            pltpu.sync_copy(idx_h.at[pl.ds(base + off, CH)], idx_v)
            pltpu.sync_copy(tp_h.at[pl.ds(base + off, CH)], rows_v)
            pltpu.sync_copy(rows_v, shared.at[idx_v], add=True)

        plsc.subcore_barrier()
        pltpu.sync_copy(shared.at[pl.ds(si * rps, rps)],
                        out_h.at[ci].at[pl.ds(si * rps, rps)])

    return k(tp, idx, zeros_tab)


def _node_encode(node_attr, p):
    def body(na, ew, eb, w1, b1, w2, b2, xf, xs):
        h = _mm(na[...], ew[...]) + eb[...]
        h = jnp.maximum(_mm(h, w1[...]) + b1[...], 0.0)
        h = _mm(h, w2[...]) + b2[...]
        xs[...] = h
        xf[...] = jnp.concatenate([h, jnp.zeros((NB, 16), F32)], axis=1)

    full = lambda a: pl.BlockSpec(a.shape, lambda i: (0,) * a.ndim)
    args = (node_attr, p['enc_w'], p['enc_b'].reshape(1, NS),
            p['ne_w1'], p['ne_b1'].reshape(1, NS),
            p['ne_w2'], p['ne_b2'].reshape(1, NS))
    return pl.pallas_call(
        body,
        grid=(N_NODES // NB,),
        in_specs=[pl.BlockSpec((NB, 128), lambda i: (i, 0))] + [full(a) for a in args[1:]],
        out_specs=[pl.BlockSpec((NB, 32), lambda i: (i, 0)),
                   pl.BlockSpec((NB, 16), lambda i: (i, 0))],
        out_shape=[jax.ShapeDtypeStruct((N_NODES, 32), F32),
                   jax.ShapeDtypeStruct((N_NODES, 16), F32)],
    )(*args)


def _edge_pre(edge_attr, psrc, pdst, p):
    offs = jnp.asarray(_RBF_OFFS.reshape(R_EMB_DIM, 1))

    def body(ea, ps, pd, ew1, eb1, ew2, eb2, rw1, rb1, rw2, rb2, outT):
        i = pl.program_id(0)

        @pl.when(i < NREAL)
        def _():
            eT = _tt(ea[...])
            psT = _tt(ps[...])
            pdT = _tt(pd[...])
            ev = psT[0:3] - pdT[0:3]
            d = jnp.sqrt(ev[0:1] * ev[0:1] + ev[1:2] * ev[1:2]
                         + ev[2:3] * ev[2:3] + 1e-12)
            emb = jnp.exp(_RBF_COEFF * (d - offs) ** 2)
            h1 = jnp.maximum(_mm(ew1[...], eT) + eb1[...], 0.0)
            e1 = _mm(ew2[...], h1) + eb2[...]
            h2 = jnp.maximum(_mm(rw1[...], emb) + rb1[...], 0.0)
            e2 = _mm(rw2[...], h2) + rb2[...]
            inv = 1.0 / d
            shv = _SQ3 * jnp.concatenate(
                [ev[1:2] * inv, ev[2:3] * inv, ev[0:1] * inv], axis=0)
            outT[...] = jnp.concatenate(
                [e1 + e2, shv, jnp.zeros((5, EB), F32)], axis=0)

        @pl.when(i >= NREAL)
        def _():
            outT[...] = jnp.zeros((24, EB), F32)

    full = lambda a: pl.BlockSpec(a.shape, lambda i: (0,) * a.ndim)
    wargs = (p['ee_w1'].T, p['ee_b1'].reshape(NS, 1),
             p['ee_w2'].T, p['ee_b2'].reshape(NS, 1),
             p['re_w1'].T, p['re_b1'].reshape(NS, 1),
             p['re_w2'].T, p['re_b2'].reshape(NS, 1))
    clamp = lambda i: (jnp.minimum(i, NREAL - 1), 0)
    return pl.pallas_call(
        body,
        grid=(NBLK,),
        in_specs=[pl.BlockSpec((EB, 16), clamp),
                  pl.BlockSpec((EB, 16), lambda i: (i, 0)),
                  pl.BlockSpec((EB, 16), lambda i: (i, 0))] + [full(a) for a in wargs],
        out_specs=pl.BlockSpec((24, EB), lambda i: (0, i)),
        out_shape=jax.ShapeDtypeStruct((24, EPAD), F32),
    )(edge_attr, psrc, pdst, *wargs)


def _conv(edgeT, gs, gd, w1t, b1c, w2t, b2c):
    def body(et, gs_r, gd_r, w1, b1, w2, b2, tp):
        i = pl.program_id(0)

        @pl.when(i < NREAL)
        def _():
            eT = et[0:16]
            shv = et[16:19]
            gsT = _tt(gs_r[...])
            gdT = _tt(gd_r[...])
            s_in = gdT[0:16]
            v_in = gdT[16:28]
            eaT = jnp.concatenate([eT, gsT, s_in], axis=0)
            z = jnp.maximum(_mm(w1[...], eaT) + b1[...], 0.0)
            w = _mm(w2[...], z) + b2[...]

            s0, s1, s2 = shv[0:1], shv[1:2], shv[2:3]
            os_ = w[0:16] * s_in[0:1]
            for u in range(1, 16):
                os_ = os_ + w[16 * u:16 * u + 16] * s_in[u:u + 1]
            for u in range(4):
                dv = (v_in[3 * u:3 * u + 1] * s0
                      + v_in[3 * u + 1:3 * u + 2] * s1
                      + v_in[3 * u + 2:3 * u + 3] * s2)
                os_ = os_ + w[256 + 16 * u:256 + 16 * u + 16] * dv
            t011 = w[320:324] * s_in[0:1]
            for u in range(1, 16):
                t011 = t011 + w[320 + 4 * u:324 + 4 * u] * s_in[u:u + 1]
            crs = []
            for u in range(4):
                vx = v_in[3 * u:3 * u + 1]
                vy = v_in[3 * u + 1:3 * u + 2]
                vz = v_in[3 * u + 2:3 * u + 3]
                crs += [vy * s2 - vz * s1, vz * s0 - vx * s2, vx * s1 - vy * s0]
            cr = jnp.concatenate(crs, axis=0)
            ovp = []
            for wo in range(4):
                acc = t011[wo:wo + 1] * shv
                for u in range(4):
                    acc = acc + w[384 + 4 * u + wo:385 + 4 * u + wo] * v_in[3 * u:3 * u + 3]
                    acc = acc + w[400 + 4 * u + wo:401 + 4 * u + wo] * cr[3 * u:3 * u + 3]
                ovp.append(acc)
            tpT = jnp.concatenate(
                [os_] + ovp + [jnp.ones((1, EB), F32), jnp.zeros((3, EB), F32)],
                axis=0)
            tp[...] = lax.dot_general(tpT, jnp.eye(32, dtype=F32),
                                      (((0,), (0,)), ((), ())),
                                      preferred_element_type=F32)

        @pl.when(i >= NREAL)
        def _():
            tp[...] = jnp.zeros((EB, 32), F32)

    full = lambda a: pl.BlockSpec(a.shape, lambda i: (0,) * a.ndim)
    return pl.pallas_call(
        body,
        grid=(NBLK,),
        in_specs=[pl.BlockSpec((24, EB), lambda i: (0, i)),
                  pl.BlockSpec((EB, 16), lambda i: (i, 0)),
                  pl.BlockSpec((EB, 32), lambda i: (i, 0)),
                  full(w1t), full(b1c), full(w2t), full(b2c)],
        out_specs=pl.BlockSpec((EB, 32), lambda i: (i, 0)),
        out_shape=jax.ShapeDtypeStruct((EPAD, 32), F32),
    )(edgeT, gs, gd, w1t, b1c, w2t, b2c)


def _update(xf, p0, p1):
    def body(x, a, b, nf, ns_):
        s = a[...] + b[...]
        cnt = jnp.maximum(s[:, 28:29], 1.0)
        xn = x[...] + jnp.concatenate(
            [s[:, :28] / cnt, jnp.zeros((NB, 4), F32)], axis=1)
        nf[...] = xn
        ns_[...] = xn[:, :16]

    bs = pl.BlockSpec((NB, 32), lambda i: (i, 0))
    return pl.pallas_call(
        body,
        grid=(N_NODES // NB,),
        in_specs=[bs, bs, bs],
        out_specs=[bs, pl.BlockSpec((NB, 16), lambda i: (i, 0))],
        out_shape=[jax.ShapeDtypeStruct((N_NODES, 32), F32),
                   jax.ShapeDtypeStruct((N_NODES, 16), F32)],
    )(xf, p0, p1)


def _final(xf, p0, p1, wfin):
    def body(x, a, b, wf, out):
        s = a[...] + b[...]
        cnt = jnp.maximum(s[:, 28:29], 1.0)
        xn = x[...] + jnp.concatenate(
            [s[:, :28] / cnt, jnp.zeros((NB, 4), F32)], axis=1)
        out[...] = _mm(xn, wf[...])

    bs = pl.BlockSpec((NB, 32), lambda i: (i, 0))
    return pl.pallas_call(
        body,
        grid=(N_NODES // NB,),
        in_specs=[bs, bs, bs, pl.BlockSpec((32, 128), lambda i: (0, 0))],
        out_specs=pl.BlockSpec((NB, 128), lambda i: (i, 0)),
        out_shape=jax.ShapeDtypeStruct((N_NODES, 128), F32),
    )(xf, p0, p1, wfin)


def kernel(pos, node_attr, edge_attr, edge_index, params):
    p = params
    src = edge_index[0].astype(jnp.int32)
    dst = edge_index[1].astype(jnp.int32)
    padi = jnp.zeros((EPAD - N_EDGES,), jnp.int32)
    src_p = jnp.concatenate([src, padi])
    dst_p = jnp.concatenate([dst, padi])
    posp = jnp.pad(pos.astype(F32), ((0, 0), (0, 13)))
    zeros_tab = jnp.zeros((N_NODES, 32), F32)

    xf, xs = _node_encode(node_attr, p)
    psrc, pdst = _sc_gather_pair(posp, src_p, posp, dst_p)
    edgeT = _edge_pre(edge_attr, psrc, pdst, p)

    scal = jnp.asarray(_S416)
    out = None
    for i in range(2):
        w1t = p['fc_w1'][i].T
        b1c = p['fc_b1'][i].reshape(128, 1)
        w2t = (p['fc_w2'][i] * scal[None, :]).T
        b2c = (p['fc_b2'][i] * scal).reshape(416, 1)
        gs, gd = _sc_gather_pair(xs, src_p, xf, dst_p)
        tp = _conv(edgeT, gs, gd, w1t, b1c, w2t, b2c)
        parts = _sc_scatter_add(tp, src_p, zeros_tab)
        if i == 0:
            xf, xs = _update(xf, parts[0], parts[1])
        else:
            wv = jnp.kron(p['lf_w1'], jnp.eye(3, dtype=F32))
            wcat = jnp.zeros((32, 128), F32)
            wcat = wcat.at[0:16, 0:32].set(p['lf_w0'])
            wcat = wcat.at[16:28, 32:128].set(wv)
            wfin = wcat[:, jnp.asarray(_OUT_PERM)]
            out = _final(xf, parts[0], parts[1], wfin)
    return out.reshape(N_NODES, 4, 8, 4)


# speedup vs baseline: 175.4455x; 175.4455x over previous
import functools

import numpy as np
import jax
import jax.numpy as jnp
from jax import lax
from jax.experimental import pallas as pl
from jax.experimental.pallas import tpu as pltpu
from jax.experimental.pallas import tpu_sc as plsc

N_NODES = 10000
N_EDGES = 160000
NS, NV = 16, 4
R_EMB_DIM, R_EMB_MAX = 32, 10.0
OUT_CH = 32

EPAD = 163840
NW = 32
PER_W = EPAD // NW
CH = 128
EB = 1280
NBLK = EPAD // EB
NREAL = N_EDGES // EB
NB = 2000

F32 = jnp.float32
_SQ3 = float(np.sqrt(3.0))
_A0 = 1.0 / float(np.sqrt(NS + NV))
_A1 = 1.0 / float(np.sqrt(NS + 2 * NV))

_S416 = np.ones((416,), np.float32)
_S416[0:256] = _A0
_S416[256:320] = _A0 / _SQ3
_S416[320:384] = _A1
_S416[384:400] = _A1
_S416[400:416] = _A1 / float(np.sqrt(2.0))

_IDXS = np.concatenate([np.arange(OUT_CH), np.repeat(np.arange(OUT_CH), 3)])
_ORDR = np.concatenate([np.zeros(OUT_CH, np.int64), np.tile(np.arange(1, 4), OUT_CH)])
_OUT_PERM = np.argsort(_IDXS * 4 + _ORDR, kind="stable")

_RBF_OFFS = np.linspace(0.0, R_EMB_MAX, R_EMB_DIM).astype(np.float32)
_RBF_COEFF = float(-0.5 / (_RBF_OFFS[1] - _RBF_OFFS[0]) ** 2)


def _tt(x):
    eye = jnp.eye(x.shape[1], dtype=F32)
    return lax.dot_general(eye, x, (((1,), (1,)), ((), ())),
                           preferred_element_type=F32)


def _mm(a, b):
    return jnp.dot(a, b, preferred_element_type=F32)


def _sc_mesh():
    return plsc.VectorSubcoreMesh(core_axis_name="c", subcore_axis_name="s")


def _sc_gather_pair(tab_a, idx_a, tab_b, idx_b):
    wa, wb = tab_a.shape[1], tab_b.shape[1]

    @functools.partial(
        pl.kernel,
        mesh=_sc_mesh(),
        compiler_params=pltpu.CompilerParams(use_tc_tiling_on_sc=False),
        out_type=(jax.ShapeDtypeStruct((EPAD, wa), F32),
                  jax.ShapeDtypeStruct((EPAD, wb), F32)),
        scratch_types=[pltpu.VMEM((CH,), jnp.int32),
                       pltpu.VMEM((CH,), jnp.int32),
                       pltpu.VMEM((CH, wa), F32),
                       pltpu.VMEM((CH, wb), F32),
                       pltpu.SemaphoreType.DMA,
                       pltpu.SemaphoreType.DMA],
    )
    def k(ta, ia_h, tb, ib_h, oa, ob, ia_v, ib_v, ra, rb, s1, s2):
        wid = lax.axis_index("s") * 2 + lax.axis_index("c")
        base = wid * PER_W

        @pl.loop(0, PER_W, step=CH)
        def _(off):
            pltpu.sync_copy(ia_h.at[pl.ds(base + off, CH)], ia_v)
            pltpu.sync_copy(ib_h.at[pl.ds(base + off, CH)], ib_v)
            c1 = pltpu.async_copy(ta.at[ia_v], ra, s1)
            c2 = pltpu.async_copy(tb.at[ib_v], rb, s2)
            c1.wait()
            c2.wait()
            pltpu.sync_copy(ra, oa.at[pl.ds(base + off, CH)])
            pltpu.sync_copy(rb, ob.at[pl.ds(base + off, CH)])

    return k(tab_a, idx_a, tab_b, idx_b)


def _sc_scatter_add(tp, idx, zeros_tab):
    rps = N_NODES // 16

    @functools.partial(
        pl.kernel,
        mesh=_sc_mesh(),
        compiler_params=pltpu.CompilerParams(use_tc_tiling_on_sc=False),
        out_type=jax.ShapeDtypeStruct((2, N_NODES, 32), F32),
        scratch_types=[pltpu.VMEM((CH,), jnp.int32),
                       pltpu.VMEM((CH, 32), F32),
                       pltpu.VMEM_SHARED((N_NODES, 32), F32)],
    )
    def k(tp_h, idx_h, z_h, out_h, idx_v, rows_v, shared):
        ci = lax.axis_index("c")
        si = lax.axis_index("s")
        wid = si * 2 + ci
        pltpu.sync_copy(z_h.at[pl.ds(si * rps, rps)],
                        shared.at[pl.ds(si * rps, rps)])
        plsc.subcore_barrier()
        base = wid * PER_W

        @pl.loop(0, PER_W, step=CH)
        def _(off):
            pltpu.sync_copy(idx_h.at[pl.ds(base + off, CH)], idx_v)
            pltpu.sync_copy(tp_h.at[pl.ds(base + off, CH)], rows_v)
            pltpu.sync_copy(rows_v, shared.at[idx_v], add=True)

        plsc.subcore_barrier()
        pltpu.sync_copy(shared.at[pl.ds(si * rps, rps)],
                        out_h.at[ci].at[pl.ds(si * rps, rps)])

    return k(tp, idx, zeros_tab)


def _node_encode(node_attr, p):
    def body(na, ew, eb, w1, b1, w2, b2, xf, xs):
        h = _mm(na[...], ew[...]) + eb[...]
        h = jnp.maximum(_mm(h, w1[...]) + b1[...], 0.0)
        h = _mm(h, w2[...]) + b2[...]
        xs[...] = h
        xf[...] = jnp.concatenate([h, jnp.zeros((NB, 16), F32)], axis=1)

    full = lambda a: pl.BlockSpec(a.shape, lambda i: (0,) * a.ndim)
    args = (node_attr, p['enc_w'], p['enc_b'].reshape(1, NS),
            p['ne_w1'], p['ne_b1'].reshape(1, NS),
            p['ne_w2'], p['ne_b2'].reshape(1, NS))
    return pl.pallas_call(
        body,
        grid=(N_NODES // NB,),
        in_specs=[pl.BlockSpec((NB, 128), lambda i: (i, 0))] + [full(a) for a in args[1:]],
        out_specs=[pl.BlockSpec((NB, 32), lambda i: (i, 0)),
                   pl.BlockSpec((NB, 16), lambda i: (i, 0))],
        out_shape=[jax.ShapeDtypeStruct((N_NODES, 32), F32),
                   jax.ShapeDtypeStruct((N_NODES, 16), F32)],
    )(*args)


def _edge_pre(edge_attr, psrc, pdst, p):
    step = float(R_EMB_MAX / (R_EMB_DIM - 1))

    def body(ea, ps, pd, ew1, eb1, ew2, eb2, rw1, rb1, rw2, rb2, outT):
        i = pl.program_id(0)
        offs = step * lax.broadcasted_iota(
            jnp.int32, (R_EMB_DIM, 1), 0).astype(F32)

        @pl.when(i < NREAL)
        def _():
            eT = _tt(ea[...])
            psT = _tt(ps[...])
            pdT = _tt(pd[...])
            ev = psT[0:3] - pdT[0:3]
            d = jnp.sqrt(ev[0:1] * ev[0:1] + ev[1:2] * ev[1:2]
                         + ev[2:3] * ev[2:3] + 1e-12)
            emb = jnp.exp(_RBF_COEFF * (d - offs) ** 2)
            h1 = jnp.maximum(_mm(ew1[...], eT) + eb1[...], 0.0)
            e1 = _mm(ew2[...], h1) + eb2[...]
            h2 = jnp.maximum(_mm(rw1[...], emb) + rb1[...], 0.0)
            e2 = _mm(rw2[...], h2) + rb2[...]
            inv = 1.0 / d
            shv = _SQ3 * jnp.concatenate(
                [ev[1:2] * inv, ev[2:3] * inv, ev[0:1] * inv], axis=0)
            outT[...] = jnp.concatenate(
                [e1 + e2, shv, jnp.zeros((5, EB), F32)], axis=0)

        @pl.when(i >= NREAL)
        def _():
            outT[...] = jnp.zeros((24, EB), F32)

    full = lambda a: pl.BlockSpec(a.shape, lambda i: (0,) * a.ndim)
    wargs = (p['ee_w1'].T, p['ee_b1'].reshape(NS, 1),
             p['ee_w2'].T, p['ee_b2'].reshape(NS, 1),
             p['re_w1'].T, p['re_b1'].reshape(NS, 1),
             p['re_w2'].T, p['re_b2'].reshape(NS, 1))
    clamp = lambda i: (jnp.minimum(i, NREAL - 1), 0)
    return pl.pallas_call(
        body,
        grid=(NBLK,),
        in_specs=[pl.BlockSpec((EB, 16), clamp),
                  pl.BlockSpec((EB, 16), lambda i: (i, 0)),
                  pl.BlockSpec((EB, 16), lambda i: (i, 0))] + [full(a) for a in wargs],
        out_specs=pl.BlockSpec((24, EB), lambda i: (0, i)),
        out_shape=jax.ShapeDtypeStruct((24, EPAD), F32),
    )(edge_attr, psrc, pdst, *wargs)


def _conv(edgeT, gs, gd, w1t, b1c, w2t, b2c):
    def body(et, gs_r, gd_r, w1, b1, w2, b2, tp):
        i = pl.program_id(0)

        @pl.when(i < NREAL)
        def _():
            eT = et[0:16]
            shv = et[16:19]
            gsT = _tt(gs_r[...])
            gdT = _tt(gd_r[...])
            s_in = gdT[0:16]
            v_in = gdT[16:28]
            eaT = jnp.concatenate([eT, gsT, s_in], axis=0)
            z = jnp.maximum(_mm(w1[...], eaT) + b1[...], 0.0)
            w = _mm(w2[...], z) + b2[...]

            s0, s1, s2 = shv[0:1], shv[1:2], shv[2:3]
            os_ = w[0:16] * s_in[0:1]
            for u in range(1, 16):
                os_ = os_ + w[16 * u:16 * u + 16] * s_in[u:u + 1]
            for u in range(4):
                dv = (v_in[3 * u:3 * u + 1] * s0
                      + v_in[3 * u + 1:3 * u + 2] * s1
                      + v_in[3 * u + 2:3 * u + 3] * s2)
                os_ = os_ + w[256 + 16 * u:256 + 16 * u + 16] * dv
            t011 = w[320:324] * s_in[0:1]
            for u in range(1, 16):
                t011 = t011 + w[320 + 4 * u:324 + 4 * u] * s_in[u:u + 1]
            crs = []
            for u in range(4):
                vx = v_in[3 * u:3 * u + 1]
                vy = v_in[3 * u + 1:3 * u + 2]
                vz = v_in[3 * u + 2:3 * u + 3]
                crs += [vy * s2 - vz * s1, vz * s0 - vx * s2, vx * s1 - vy * s0]
            cr = jnp.concatenate(crs, axis=0)
            ovp = []
            for wo in range(4):
                acc = t011[wo:wo + 1] * shv
                for u in range(4):
                    acc = acc + w[384 + 4 * u + wo:385 + 4 * u + wo] * v_in[3 * u:3 * u + 3]
                    acc = acc + w[400 + 4 * u + wo:401 + 4 * u + wo] * cr[3 * u:3 * u + 3]
                ovp.append(acc)
            tpT = jnp.concatenate(
                [os_] + ovp + [jnp.ones((1, EB), F32), jnp.zeros((3, EB), F32)],
                axis=0)
            tp[...] = lax.dot_general(tpT, jnp.eye(32, dtype=F32),
                                      (((0,), (0,)), ((), ())),
                                      preferred_element_type=F32)

        @pl.when(i >= NREAL)
        def _():
            tp[...] = jnp.zeros((EB, 32), F32)

    full = lambda a: pl.BlockSpec(a.shape, lambda i: (0,) * a.ndim)
    return pl.pallas_call(
        body,
        grid=(NBLK,),
        in_specs=[pl.BlockSpec((24, EB), lambda i: (0, i)),
                  pl.BlockSpec((EB, 16), lambda i: (i, 0)),
                  pl.BlockSpec((EB, 32), lambda i: (i, 0)),
                  full(w1t), full(b1c), full(w2t), full(b2c)],
        out_specs=pl.BlockSpec((EB, 32), lambda i: (i, 0)),
        out_shape=jax.ShapeDtypeStruct((EPAD, 32), F32),
    )(edgeT, gs, gd, w1t, b1c, w2t, b2c)


def _update(xf, p0, p1):
    def body(x, a, b, nf, ns_):
        s = a[...] + b[...]
        cnt = jnp.maximum(s[:, 28:29], 1.0)
        xn = x[...] + jnp.concatenate(
            [s[:, :28] / cnt, jnp.zeros((NB, 4), F32)], axis=1)
        nf[...] = xn
        ns_[...] = xn[:, :16]

    bs = pl.BlockSpec((NB, 32), lambda i: (i, 0))
    return pl.pallas_call(
        body,
        grid=(N_NODES // NB,),
        in_specs=[bs, bs, bs],
        out_specs=[bs, pl.BlockSpec((NB, 16), lambda i: (i, 0))],
        out_shape=[jax.ShapeDtypeStruct((N_NODES, 32), F32),
                   jax.ShapeDtypeStruct((N_NODES, 16), F32)],
    )(xf, p0, p1)


def _final(xf, p0, p1, wfin):
    def body(x, a, b, wf, out):
        s = a[...] + b[...]
        cnt = jnp.maximum(s[:, 28:29], 1.0)
        xn = x[...] + jnp.concatenate(
            [s[:, :28] / cnt, jnp.zeros((NB, 4), F32)], axis=1)
        out[...] = _mm(xn, wf[...])

    bs = pl.BlockSpec((NB, 32), lambda i: (i, 0))
    return pl.pallas_call(
        body,
        grid=(N_NODES // NB,),
        in_specs=[bs, bs, bs, pl.BlockSpec((32, 128), lambda i: (0, 0))],
        out_specs=pl.BlockSpec((NB, 128), lambda i: (i, 0)),
        out_shape=jax.ShapeDtypeStruct((N_NODES, 128), F32),
    )(xf, p0, p1, wfin)


def kernel(pos, node_attr, edge_attr, edge_index, params):
    p = params
    src = edge_index[0].astype(jnp.int32)
    dst = edge_index[1].astype(jnp.int32)
    padi = jnp.zeros((EPAD - N_EDGES,), jnp.int32)
    src_p = jnp.concatenate([src, padi])
    dst_p = jnp.concatenate([dst, padi])
    posp = jnp.pad(pos.astype(F32), ((0, 0), (0, 13)))
    zeros_tab = jnp.zeros((N_NODES, 32), F32)

    xf, xs = _node_encode(node_attr, p)
    psrc, pdst = _sc_gather_pair(posp, src_p, posp, dst_p)
    edgeT = _edge_pre(edge_attr, psrc, pdst, p)

    scal = jnp.asarray(_S416)
    out = None
    for i in range(2):
        w1t = p['fc_w1'][i].T
        b1c = p['fc_b1'][i].reshape(128, 1)
        w2t = (p['fc_w2'][i] * scal[None, :]).T
        b2c = (p['fc_b2'][i] * scal).reshape(416, 1)
        gs, gd = _sc_gather_pair(xs, src_p, xf, dst_p)
        tp = _conv(edgeT, gs, gd, w1t, b1c, w2t, b2c)
        parts = _sc_scatter_add(tp, src_p, zeros_tab)
        if i == 0:
            xf, xs = _update(xf, parts[0], parts[1])
        else:
            wv = jnp.kron(p['lf_w1'], jnp.eye(3, dtype=F32))
            wcat = jnp.zeros((32, 128), F32)
            wcat = wcat.at[0:16, 0:32].set(p['lf_w0'])
            wcat = wcat.at[16:28, 32:128].set(wv)
            wfin = wcat[:, jnp.asarray(_OUT_PERM)]
            out = _final(xf, parts[0], parts[1], wfin)
    return out.reshape(N_NODES, 4, 8, 4)


# speedup vs baseline: 204.6899x; 1.1667x over previous
import functools

import numpy as np
import jax
import jax.numpy as jnp
from jax import lax
from jax.experimental import pallas as pl
from jax.experimental.pallas import tpu as pltpu
from jax.experimental.pallas import tpu_sc as plsc

N_NODES = 10000
N_EDGES = 160000
NS, NV = 16, 4
R_EMB_DIM, R_EMB_MAX = 32, 10.0
OUT_CH = 32

EPAD = 163840
NW = 32
PER_W = EPAD // NW
CH = 128
EB = 1280
NBLK = EPAD // EB
NREAL = N_EDGES // EB
NB = 2000

F32 = jnp.float32
_SQ3 = float(np.sqrt(3.0))
_A0 = 1.0 / float(np.sqrt(NS + NV))
_A1 = 1.0 / float(np.sqrt(NS + 2 * NV))

_S416 = np.ones((416,), np.float32)
_S416[0:256] = _A0
_S416[256:320] = _A0 / _SQ3
_S416[320:384] = _A1
_S416[384:400] = _A1
_S416[400:416] = _A1 / float(np.sqrt(2.0))

_IDXS = np.concatenate([np.arange(OUT_CH), np.repeat(np.arange(OUT_CH), 3)])
_ORDR = np.concatenate([np.zeros(OUT_CH, np.int64), np.tile(np.arange(1, 4), OUT_CH)])
_OUT_PERM = np.argsort(_IDXS * 4 + _ORDR, kind="stable")

_RBF_OFFS = np.linspace(0.0, R_EMB_MAX, R_EMB_DIM).astype(np.float32)
_RBF_COEFF = float(-0.5 / (_RBF_OFFS[1] - _RBF_OFFS[0]) ** 2)


def _tt(x):
    eye = jnp.eye(x.shape[1], dtype=F32)
    return lax.dot_general(eye, x, (((1,), (1,)), ((), ())),
                           preferred_element_type=F32)


def _mm(a, b):
    return jnp.dot(a, b, preferred_element_type=F32)


def _sc_mesh():
    return plsc.VectorSubcoreMesh(core_axis_name="c", subcore_axis_name="s")


def _sc_gather_pair(tab_a, idx_a, tab_b, idx_b):
    wa, wb = tab_a.shape[1], tab_b.shape[1]

    nch = PER_W // CH

    @functools.partial(
        pl.kernel,
        mesh=_sc_mesh(),
        compiler_params=pltpu.CompilerParams(use_tc_tiling_on_sc=False),
        out_type=(jax.ShapeDtypeStruct((EPAD, wa), F32),
                  jax.ShapeDtypeStruct((EPAD, wb), F32)),
        scratch_types=[pltpu.VMEM((PER_W,), jnp.int32),
                       pltpu.VMEM((PER_W,), jnp.int32),
                       pltpu.VMEM((CH, wa), F32),
                       pltpu.VMEM((CH, wa), F32),
                       pltpu.VMEM((CH, wb), F32),
                       pltpu.VMEM((CH, wb), F32),
                       pltpu.SemaphoreType.DMA,
                       pltpu.SemaphoreType.DMA,
                       pltpu.SemaphoreType.DMA,
                       pltpu.SemaphoreType.DMA],
    )
    def k(ta, ia_h, tb, ib_h, oa, ob, ia_v, ib_v,
          ra0, ra1, rb0, rb1, sa0, sa1, sb0, sb1):
        wid = lax.axis_index("s") * 2 + lax.axis_index("c")
        base = wid * PER_W
        pltpu.sync_copy(ia_h.at[pl.ds(base, PER_W)], ia_v)
        pltpu.sync_copy(ib_h.at[pl.ds(base, PER_W)], ib_v)
        pltpu.async_copy(ta.at[ia_v.at[pl.ds(0, CH)]], ra0, sa0)
        pltpu.async_copy(tb.at[ib_v.at[pl.ds(0, CH)]], rb0, sb0)

        @pl.loop(0, nch, step=2)
        def _(g):
            off = g * CH
            c1 = pltpu.async_copy(ta.at[ia_v.at[pl.ds(off + CH, CH)]], ra1, sa1)
            c2 = pltpu.async_copy(tb.at[ib_v.at[pl.ds(off + CH, CH)]], rb1, sb1)
            pltpu.make_async_copy(ta.at[pl.ds(0, CH)], ra0, sa0).wait()
            pltpu.make_async_copy(tb.at[pl.ds(0, CH)], rb0, sb0).wait()
            pltpu.sync_copy(ra0, oa.at[pl.ds(base + off, CH)])
            pltpu.sync_copy(rb0, ob.at[pl.ds(base + off, CH)])

            @pl.when(g + 2 < nch)
            def _():
                pltpu.async_copy(ta.at[ia_v.at[pl.ds(off + 2 * CH, CH)]], ra0, sa0)
                pltpu.async_copy(tb.at[ib_v.at[pl.ds(off + 2 * CH, CH)]], rb0, sb0)

            c1.wait()
            c2.wait()
            pltpu.sync_copy(ra1, oa.at[pl.ds(base + off + CH, CH)])
            pltpu.sync_copy(rb1, ob.at[pl.ds(base + off + CH, CH)])

    return k(tab_a, idx_a, tab_b, idx_b)


def _sc_scatter_add(tp, idx, zeros_tab):
    rps = N_NODES // 16

    @functools.partial(
        pl.kernel,
        mesh=_sc_mesh(),
        compiler_params=pltpu.CompilerParams(use_tc_tiling_on_sc=False),
        out_type=jax.ShapeDtypeStruct((2, N_NODES, 32), F32),
        scratch_types=[pltpu.VMEM((CH,), jnp.int32),
                       pltpu.VMEM((CH,), jnp.int32),
                       pltpu.VMEM((CH, 32), F32),
                       pltpu.VMEM((CH, 32), F32),
                       pltpu.VMEM_SHARED((N_NODES, 32), F32),
                       pltpu.SemaphoreType.DMA,
                       pltpu.SemaphoreType.DMA],
    )
    def k(tp_h, idx_h, z_h, out_h, ia0, ia1, r0, r1, shared, s0, s1):
        ci = lax.axis_index("c")
        si = lax.axis_index("s")
        wid = si * 2 + ci
        pltpu.sync_copy(z_h.at[pl.ds(si * rps, rps)],
                        shared.at[pl.ds(si * rps, rps)])
        plsc.subcore_barrier()
        base = wid * PER_W
        nch = PER_W // CH
        pltpu.sync_copy(idx_h.at[pl.ds(base, CH)], ia0)
        pltpu.async_copy(tp_h.at[pl.ds(base, CH)], r0, s0)

        @pl.loop(0, nch, step=2)
        def _(g):
            off = base + g * CH
            pltpu.sync_copy(idx_h.at[pl.ds(off + CH, CH)], ia1)
            c1 = pltpu.async_copy(tp_h.at[pl.ds(off + CH, CH)], r1, s1)
            pltpu.make_async_copy(tp_h.at[pl.ds(0, CH)], r0, s0).wait()
            pltpu.sync_copy(r0, shared.at[ia0], add=True)

            @pl.when(g + 2 < nch)
            def _():
                pltpu.sync_copy(idx_h.at[pl.ds(off + 2 * CH, CH)], ia0)
                pltpu.async_copy(tp_h.at[pl.ds(off + 2 * CH, CH)], r0, s0)

            c1.wait()
            pltpu.sync_copy(r1, shared.at[ia1], add=True)

        plsc.subcore_barrier()
        pltpu.sync_copy(shared.at[pl.ds(si * rps, rps)],
                        out_h.at[ci].at[pl.ds(si * rps, rps)])

    return k(tp, idx, zeros_tab)


def _node_encode(node_attr, p):
    def body(na, ew, eb, w1, b1, w2, b2, xf, xs):
        h = _mm(na[...], ew[...]) + eb[...]
        h = jnp.maximum(_mm(h, w1[...]) + b1[...], 0.0)
        h = _mm(h, w2[...]) + b2[...]
        xs[...] = h
        xf[...] = jnp.concatenate([h, jnp.zeros((NB, 16), F32)], axis=1)

    full = lambda a: pl.BlockSpec(a.shape, lambda i: (0,) * a.ndim)
    args = (node_attr, p['enc_w'], p['enc_b'].reshape(1, NS),
            p['ne_w1'], p['ne_b1'].reshape(1, NS),
            p['ne_w2'], p['ne_b2'].reshape(1, NS))
    return pl.pallas_call(
        body,
        grid=(N_NODES // NB,),
        in_specs=[pl.BlockSpec((NB, 128), lambda i: (i, 0))] + [full(a) for a in args[1:]],
        out_specs=[pl.BlockSpec((NB, 32), lambda i: (i, 0)),
                   pl.BlockSpec((NB, 16), lambda i: (i, 0))],
        out_shape=[jax.ShapeDtypeStruct((N_NODES, 32), F32),
                   jax.ShapeDtypeStruct((N_NODES, 16), F32)],
    )(*args)


def _edge_pre(edge_attr, psrc, pdst, p):
    step = float(R_EMB_MAX / (R_EMB_DIM - 1))

    def body(ea, ps, pd, ew1, eb1, ew2, eb2, rw1, rb1, rw2, rb2, outT):
        i = pl.program_id(0)
        offs = step * lax.broadcasted_iota(
            jnp.int32, (R_EMB_DIM, 1), 0).astype(F32)

        @pl.when(i < NREAL)
        def _():
            eT = _tt(ea[...])
            psT = _tt(ps[...])
            pdT = _tt(pd[...])
            ev = psT[0:3] - pdT[0:3]
            d = jnp.sqrt(ev[0:1] * ev[0:1] + ev[1:2] * ev[1:2]
                         + ev[2:3] * ev[2:3] + 1e-12)
            emb = jnp.exp(_RBF_COEFF * (d - offs) ** 2)
            h1 = jnp.maximum(_mm(ew1[...], eT) + eb1[...], 0.0)
            e1 = _mm(ew2[...], h1) + eb2[...]
            h2 = jnp.maximum(_mm(rw1[...], emb) + rb1[...], 0.0)
            e2 = _mm(rw2[...], h2) + rb2[...]
            inv = 1.0 / d
            shv = _SQ3 * jnp.concatenate(
                [ev[1:2] * inv, ev[2:3] * inv, ev[0:1] * inv], axis=0)
            outT[...] = jnp.concatenate(
                [e1 + e2, shv, jnp.zeros((5, EB), F32)], axis=0)

        @pl.when(i >= NREAL)
        def _():
            outT[...] = jnp.zeros((24, EB), F32)

    full = lambda a: pl.BlockSpec(a.shape, lambda i: (0,) * a.ndim)
    wargs = (p['ee_w1'].T, p['ee_b1'].reshape(NS, 1),
             p['ee_w2'].T, p['ee_b2'].reshape(NS, 1),
             p['re_w1'].T, p['re_b1'].reshape(NS, 1),
             p['re_w2'].T, p['re_b2'].reshape(NS, 1))
    clamp = lambda i: (jnp.minimum(i, NREAL - 1), 0)
    return pl.pallas_call(
        body,
        grid=(NBLK,),
        in_specs=[pl.BlockSpec((EB, 16), clamp),
                  pl.BlockSpec((EB, 16), lambda i: (i, 0)),
                  pl.BlockSpec((EB, 16), lambda i: (i, 0))] + [full(a) for a in wargs],
        out_specs=pl.BlockSpec((24, EB), lambda i: (0, i)),
        out_shape=jax.ShapeDtypeStruct((24, EPAD), F32),
    )(edge_attr, psrc, pdst, *wargs)


def _conv(edgeT, gs, gd, w1t, b1c, w2t, b2c, first):
    gw = gd.shape[1]
    t0 = 256 if first else 320

    def body(et, gs_r, gd_r, w1, b1, w2, b2, tp):
        i = pl.program_id(0)

        @pl.when(i < NREAL)
        def _():
            eT = et[0:16]
            shv = et[16:19]
            gsT = _tt(gs_r[...])
            gdT = _tt(gd_r[...])
            s_in = gdT[0:16]
            eaT = jnp.concatenate([eT, gsT, s_in], axis=0)
            z = jnp.maximum(
                jnp.dot(w1[...], eaT.astype(jnp.bfloat16),
                        preferred_element_type=F32) + b1[...], 0.0)
            w = jnp.dot(w2[...], z.astype(jnp.bfloat16),
                        preferred_element_type=F32) + b2[...]

            s0, s1, s2 = shv[0:1], shv[1:2], shv[2:3]
            os_ = w[0:16] * s_in[0:1]
            for u in range(1, 16):
                os_ = os_ + w[16 * u:16 * u + 16] * s_in[u:u + 1]
            t011 = w[t0:t0 + 4] * s_in[0:1]
            for u in range(1, 16):
                t011 = t011 + w[t0 + 4 * u:t0 + 4 * u + 4] * s_in[u:u + 1]
            if not first:
                v_in = gdT[16:28]
                for u in range(4):
                    dv = (v_in[3 * u:3 * u + 1] * s0
                          + v_in[3 * u + 1:3 * u + 2] * s1
                          + v_in[3 * u + 2:3 * u + 3] * s2)
                    os_ = os_ + w[256 + 16 * u:256 + 16 * u + 16] * dv
                crs = []
                for u in range(4):
                    vx = v_in[3 * u:3 * u + 1]
                    vy = v_in[3 * u + 1:3 * u + 2]
                    vz = v_in[3 * u + 2:3 * u + 3]
                    crs += [vy * s2 - vz * s1, vz * s0 - vx * s2,
                            vx * s1 - vy * s0]
                cr = jnp.concatenate(crs, axis=0)
            ovp = []
            for wo in range(4):
                acc = t011[wo:wo + 1] * shv
                if not first:
                    for u in range(4):
                        acc = acc + w[384 + 4 * u + wo:385 + 4 * u + wo] * v_in[3 * u:3 * u + 3]
                        acc = acc + w[400 + 4 * u + wo:401 + 4 * u + wo] * cr[3 * u:3 * u + 3]
                ovp.append(acc)
            tpT = jnp.concatenate(
                [os_] + ovp + [jnp.ones((1, EB), F32), jnp.zeros((3, EB), F32)],
                axis=0)
            tp[...] = lax.dot_general(tpT, jnp.eye(32, dtype=F32),
                                      (((0,), (0,)), ((), ())),
                                      preferred_element_type=F32)

        @pl.when(i >= NREAL)
        def _():
            tp[...] = jnp.zeros((EB, 32), F32)

    full = lambda a: pl.BlockSpec(a.shape, lambda i: (0,) * a.ndim)
    return pl.pallas_call(
        body,
        grid=(NBLK,),
        in_specs=[pl.BlockSpec((24, EB), lambda i: (0, i)),
                  pl.BlockSpec((EB, 16), lambda i: (i, 0)),
                  pl.BlockSpec((EB, gw), lambda i: (i, 0)),
                  full(w1t), full(b1c), full(w2t), full(b2c)],
        out_specs=pl.BlockSpec((EB, 32), lambda i: (i, 0)),
        out_shape=jax.ShapeDtypeStruct((EPAD, 32), F32),
    )(edgeT, gs, gd, w1t, b1c, w2t, b2c)


def _update(xf, p0, p1):
    def body(x, a, b, nf, ns_):
        s = a[...] + b[...]
        cnt = jnp.maximum(s[:, 28:29], 1.0)
        xn = x[...] + jnp.concatenate(
            [s[:, :28] / cnt, jnp.zeros((NB, 4), F32)], axis=1)
        nf[...] = xn
        ns_[...] = xn[:, :16]

    bs = pl.BlockSpec((NB, 32), lambda i: (i, 0))
    return pl.pallas_call(
        body,
        grid=(N_NODES // NB,),
        in_specs=[bs, bs, bs],
        out_specs=[bs, pl.BlockSpec((NB, 16), lambda i: (i, 0))],
        out_shape=[jax.ShapeDtypeStruct((N_NODES, 32), F32),
                   jax.ShapeDtypeStruct((N_NODES, 16), F32)],
    )(xf, p0, p1)


def _final(xf, p0, p1, wfin):
    def body(x, a, b, wf, out):
        s = a[...] + b[...]
        cnt = jnp.maximum(s[:, 28:29], 1.0)
        xn = x[...] + jnp.concatenate(
            [s[:, :28] / cnt, jnp.zeros((NB, 4), F32)], axis=1)
        out[...] = _mm(xn, wf[...])

    bs = pl.BlockSpec((NB, 32), lambda i: (i, 0))
    return pl.pallas_call(
        body,
        grid=(N_NODES // NB,),
        in_specs=[bs, bs, bs, pl.BlockSpec((32, 128), lambda i: (0, 0))],
        out_specs=pl.BlockSpec((NB, 128), lambda i: (i, 0)),
        out_shape=jax.ShapeDtypeStruct((N_NODES, 128), F32),
    )(xf, p0, p1, wfin)


def kernel(pos, node_attr, edge_attr, edge_index, params):
    p = params
    src = edge_index[0].astype(jnp.int32)
    dst = edge_index[1].astype(jnp.int32)
    padi = jnp.zeros((EPAD - N_EDGES,), jnp.int32)
    src_p = jnp.concatenate([src, padi])
    dst_p = jnp.concatenate([dst, padi])
    posp = jnp.pad(pos.astype(F32), ((0, 0), (0, 13)))
    zeros_tab = jnp.zeros((N_NODES, 32), F32)

    xf, xs = _node_encode(node_attr, p)
    psrc, pdst = _sc_gather_pair(posp, src_p, posp, dst_p)
    edgeT = _edge_pre(edge_attr, psrc, pdst, p)

    scal = jnp.asarray(_S416)
    out = None
    for i in range(2):
        first = i == 0
        w1t = p['fc_w1'][i].T.astype(jnp.bfloat16)
        b1c = p['fc_b1'][i].reshape(128, 1)
        w2t = (p['fc_w2'][i] * scal[None, :]).T
        b2c = (p['fc_b2'][i] * scal).reshape(416, 1)
        if first:
            w2t = jnp.concatenate([w2t[0:256], w2t[320:384]])
            b2c = jnp.concatenate([b2c[0:256], b2c[320:384]])
        w2t = w2t.astype(jnp.bfloat16)
        gs, gd = _sc_gather_pair(xs, src_p, xs if first else xf, dst_p)
        tp = _conv(edgeT, gs, gd, w1t, b1c, w2t, b2c, first)
        parts = _sc_scatter_add(tp, src_p, zeros_tab)
        if i == 0:
            xf, xs = _update(xf, parts[0], parts[1])
        else:
            wv = jnp.kron(p['lf_w1'], jnp.eye(3, dtype=F32))
            wcat = jnp.zeros((32, 128), F32)
            wcat = wcat.at[0:16, 0:32].set(p['lf_w0'])
            wcat = wcat.at[16:28, 32:128].set(wv)
            wfin = wcat[:, jnp.asarray(_OUT_PERM)]
            out = _final(xf, parts[0], parts[1], wfin)
    return out.reshape(N_NODES, 4, 8, 4)


# speedup vs baseline: 222.3788x; 1.0864x over previous
import functools

import numpy as np
import jax
import jax.numpy as jnp
from jax import lax
from jax.experimental import pallas as pl
from jax.experimental.pallas import tpu as pltpu
from jax.experimental.pallas import tpu_sc as plsc

N_NODES = 10000
N_EDGES = 160000
NS, NV = 16, 4
R_EMB_DIM, R_EMB_MAX = 32, 10.0
OUT_CH = 32

EPAD = 163840
NW = 32
PER_W = EPAD // NW
CH = 128
EB = 2560
NBLK = EPAD // EB
NB = 2000

F32 = jnp.float32
_SQ3 = float(np.sqrt(3.0))
_A0 = 1.0 / float(np.sqrt(NS + NV))
_A1 = 1.0 / float(np.sqrt(NS + 2 * NV))

_S416 = np.ones((416,), np.float32)
_S416[0:256] = _A0
_S416[256:320] = _A0 / _SQ3
_S416[320:384] = _A1
_S416[384:400] = _A1
_S416[400:416] = _A1 / float(np.sqrt(2.0))

_IDXS = np.concatenate([np.arange(OUT_CH), np.repeat(np.arange(OUT_CH), 3)])
_ORDR = np.concatenate([np.zeros(OUT_CH, np.int64), np.tile(np.arange(1, 4), OUT_CH)])
_OUT_PERM = np.argsort(_IDXS * 4 + _ORDR, kind="stable")

_RBF_OFFS = np.linspace(0.0, R_EMB_MAX, R_EMB_DIM).astype(np.float32)
_RBF_COEFF = float(-0.5 / (_RBF_OFFS[1] - _RBF_OFFS[0]) ** 2)


def _tt(x):
    eye = jnp.eye(x.shape[1], dtype=F32)
    return lax.dot_general(eye, x, (((1,), (1,)), ((), ())),
                           preferred_element_type=F32)


def _mm(a, b):
    return jnp.dot(a, b, preferred_element_type=F32)


def _sc_mesh():
    return plsc.VectorSubcoreMesh(core_axis_name="c", subcore_axis_name="s")


def _sc_gather_multi(tabs, sels, idx0, idx1):
    ns = len(tabs)
    ws = [t.shape[1] for t in tabs]
    nch = PER_W // CH
    scratch = [pltpu.VMEM((PER_W,), jnp.int32), pltpu.VMEM((PER_W,), jnp.int32)]
    for w in ws:
        scratch += [pltpu.VMEM((CH, w), F32), pltpu.VMEM((CH, w), F32)]
    scratch += [pltpu.SemaphoreType.DMA] * (2 * ns)

    @functools.partial(
        pl.kernel,
        mesh=_sc_mesh(),
        compiler_params=pltpu.CompilerParams(use_tc_tiling_on_sc=False),
        out_type=tuple(jax.ShapeDtypeStruct((EPAD, w), F32) for w in ws),
        scratch_types=scratch,
    )
    def k(*args):
        tabs_r = args[:ns]
        i0_h, i1_h = args[ns], args[ns + 1]
        outs = args[ns + 2:2 * ns + 2]
        i0_v, i1_v = args[2 * ns + 2], args[2 * ns + 3]
        bufs = args[2 * ns + 4:4 * ns + 4]
        sems = args[4 * ns + 4:]
        iv = (i0_v, i1_v)
        wid = lax.axis_index("s") * 2 + lax.axis_index("c")
        base = wid * PER_W
        pltpu.sync_copy(i0_h.at[pl.ds(base, PER_W)], i0_v)
        pltpu.sync_copy(i1_h.at[pl.ds(base, PER_W)], i1_v)

        def gath(s, buf_i, off, sem_i):
            pltpu.async_copy(
                tabs_r[s].at[iv[sels[s]].at[pl.ds(off, CH)]],
                bufs[2 * s + buf_i], sems[2 * s + sem_i])

        def drain(s, buf_i, sem_i):
            pltpu.make_async_copy(tabs_r[s].at[pl.ds(0, CH)],
                                  bufs[2 * s + buf_i],
                                  sems[2 * s + sem_i]).wait()

        for s in range(ns):
            gath(s, 0, 0, 0)

        @pl.loop(0, nch, step=2)
        def _(g):
            off = g * CH
            for s in range(ns):
                gath(s, 1, off + CH, 1)
            for s in range(ns):
                drain(s, 0, 0)
                pltpu.sync_copy(bufs[2 * s], outs[s].at[pl.ds(base + off, CH)])

            @pl.when(g + 2 < nch)
            def _():
                for s in range(ns):
                    gath(s, 0, off + 2 * CH, 0)

            for s in range(ns):
                drain(s, 1, 1)
                pltpu.sync_copy(bufs[2 * s + 1],
                                outs[s].at[pl.ds(base + off + CH, CH)])

    return k(*tabs, idx0, idx1)


def _sc_scatter_add(tp, idx, zeros_tab):
    rps = N_NODES // 16

    @functools.partial(
        pl.kernel,
        mesh=_sc_mesh(),
        compiler_params=pltpu.CompilerParams(use_tc_tiling_on_sc=False),
        out_type=jax.ShapeDtypeStruct((2, N_NODES, 32), F32),
        scratch_types=[pltpu.VMEM((CH,), jnp.int32),
                       pltpu.VMEM((CH,), jnp.int32),
                       pltpu.VMEM((CH, 32), F32),
                       pltpu.VMEM((CH, 32), F32),
                       pltpu.VMEM_SHARED((N_NODES, 32), F32),
                       pltpu.SemaphoreType.DMA,
                       pltpu.SemaphoreType.DMA],
    )
    def k(tp_h, idx_h, z_h, out_h, ia0, ia1, r0, r1, shared, s0, s1):
        ci = lax.axis_index("c")
        si = lax.axis_index("s")
        wid = si * 2 + ci
        pltpu.sync_copy(z_h.at[pl.ds(si * rps, rps)],
                        shared.at[pl.ds(si * rps, rps)])
        plsc.subcore_barrier()
        base = wid * PER_W
        nch = PER_W // CH
        pltpu.sync_copy(idx_h.at[pl.ds(base, CH)], ia0)
        pltpu.async_copy(tp_h.at[pl.ds(base, CH)], r0, s0)

        @pl.loop(0, nch, step=2)
        def _(g):
            off = base + g * CH
            pltpu.sync_copy(idx_h.at[pl.ds(off + CH, CH)], ia1)
            c1 = pltpu.async_copy(tp_h.at[pl.ds(off + CH, CH)], r1, s1)
            pltpu.make_async_copy(tp_h.at[pl.ds(0, CH)], r0, s0).wait()
            pltpu.sync_copy(r0, shared.at[ia0], add=True)

            @pl.when(g + 2 < nch)
            def _():
                pltpu.sync_copy(idx_h.at[pl.ds(off + 2 * CH, CH)], ia0)
                pltpu.async_copy(tp_h.at[pl.ds(off + 2 * CH, CH)], r0, s0)

            c1.wait()
            pltpu.sync_copy(r1, shared.at[ia1], add=True)

        plsc.subcore_barrier()
        pltpu.sync_copy(shared.at[pl.ds(si * rps, rps)],
                        out_h.at[ci].at[pl.ds(si * rps, rps)])

    return k(tp, idx, zeros_tab)


def _node_encode(node_attr, p):
    def body(na, ew, eb, w1, b1, w2, b2, xf, xs):
        h = _mm(na[...], ew[...]) + eb[...]
        h = jnp.maximum(_mm(h, w1[...]) + b1[...], 0.0)
        h = _mm(h, w2[...]) + b2[...]
        xs[...] = h
        xf[...] = jnp.concatenate([h, jnp.zeros((NB, 16), F32)], axis=1)

    full = lambda a: pl.BlockSpec(a.shape, lambda i: (0,) * a.ndim)
    args = (node_attr, p['enc_w'], p['enc_b'].reshape(1, NS),
            p['ne_w1'], p['ne_b1'].reshape(1, NS),
            p['ne_w2'], p['ne_b2'].reshape(1, NS))
    return pl.pallas_call(
        body,
        grid=(N_NODES // NB,),
        in_specs=[pl.BlockSpec((NB, 128), lambda i: (i, 0))] + [full(a) for a in args[1:]],
        out_specs=[pl.BlockSpec((NB, 32), lambda i: (i, 0)),
                   pl.BlockSpec((NB, 16), lambda i: (i, 0))],
        out_shape=[jax.ShapeDtypeStruct((N_NODES, 32), F32),
                   jax.ShapeDtypeStruct((N_NODES, 16), F32)],
    )(*args)


def _edge_pre(edge_attr, psrc, pdst, p):
    step = float(R_EMB_MAX / (R_EMB_DIM - 1))

    def body(ea, ps, pd, ew1, eb1, ew2, eb2, rw1, rb1, rw2, rb2, outT):
        i = pl.program_id(0)
        offs = step * lax.broadcasted_iota(
            jnp.int32, (R_EMB_DIM, 1), 0).astype(F32)
        lane = lax.broadcasted_iota(jnp.int32, (1, EB), 1) + i * EB
        eT = _tt(ea[...])
        psT = _tt(ps[...])
        pdT = _tt(pd[...])
        ev = psT[0:3] - pdT[0:3]
        d = jnp.sqrt(ev[0:1] * ev[0:1] + ev[1:2] * ev[1:2]
                     + ev[2:3] * ev[2:3] + 1e-12)
        emb = jnp.exp(_RBF_COEFF * (d - offs) ** 2)
        h1 = jnp.maximum(_mm(ew1[...], eT) + eb1[...], 0.0)
        e1 = _mm(ew2[...], h1) + eb2[...]
        h2 = jnp.maximum(_mm(rw1[...], emb) + rb1[...], 0.0)
        e2 = _mm(rw2[...], h2) + rb2[...]
        inv = 1.0 / d
        shv = _SQ3 * jnp.concatenate(
            [ev[1:2] * inv, ev[2:3] * inv, ev[0:1] * inv], axis=0)
        res = jnp.concatenate(
            [e1 + e2, shv, jnp.zeros((5, EB), F32)], axis=0)
        outT[...] = jnp.where(lane < N_EDGES, res, 0.0)

    full = lambda a: pl.BlockSpec(a.shape, lambda i: (0,) * a.ndim)
    wargs = (p['ee_w1'].T, p['ee_b1'].reshape(NS, 1),
             p['ee_w2'].T, p['ee_b2'].reshape(NS, 1),
             p['re_w1'].T, p['re_b1'].reshape(NS, 1),
             p['re_w2'].T, p['re_b2'].reshape(NS, 1))
    return pl.pallas_call(
        body,
        grid=(NBLK,),
        in_specs=[pl.BlockSpec((EB, 16), lambda i: (i, 0)),
                  pl.BlockSpec((EB, 16), lambda i: (i, 0)),
                  pl.BlockSpec((EB, 16), lambda i: (i, 0))] + [full(a) for a in wargs],
        out_specs=pl.BlockSpec((24, EB), lambda i: (0, i)),
        out_shape=jax.ShapeDtypeStruct((24, EPAD), F32),
    )(edge_attr, psrc, pdst, *wargs)


def _conv(edgeT, gs, gd, w1t, b1c, w2t, b2c, first):
    gw = gd.shape[1]
    t0 = 256 if first else 320

    def body(et, gs_r, gd_r, w1, b1, w2, b2, tp):
        i = pl.program_id(0)
        lane = lax.broadcasted_iota(jnp.int32, (1, EB), 1) + i * EB
        eT = et[0:16]
        shv = et[16:19]
        gsT = _tt(gs_r[...])
        gdT = _tt(gd_r[...])
        s_in = gdT[0:16]
        eaT = jnp.concatenate([eT, gsT, s_in], axis=0)
        z = jnp.maximum(
            jnp.dot(w1[...], eaT.astype(jnp.bfloat16),
                    preferred_element_type=F32) + b1[...], 0.0)
        w = jnp.dot(w2[...], z.astype(jnp.bfloat16),
                    preferred_element_type=F32) + b2[...]

        s0, s1, s2 = shv[0:1], shv[1:2], shv[2:3]
        os_ = w[0:16] * s_in[0:1]
        for u in range(1, 16):
            os_ = os_ + w[16 * u:16 * u + 16] * s_in[u:u + 1]
        t011 = w[t0:t0 + 4] * s_in[0:1]
        for u in range(1, 16):
            t011 = t011 + w[t0 + 4 * u:t0 + 4 * u + 4] * s_in[u:u + 1]
        if not first:
            v_in = gdT[16:28]
            for u in range(4):
                dv = (v_in[3 * u:3 * u + 1] * s0
                      + v_in[3 * u + 1:3 * u + 2] * s1
                      + v_in[3 * u + 2:3 * u + 3] * s2)
                os_ = os_ + w[256 + 16 * u:256 + 16 * u + 16] * dv
            crs = []
            for u in range(4):
                vx = v_in[3 * u:3 * u + 1]
                vy = v_in[3 * u + 1:3 * u + 2]
                vz = v_in[3 * u + 2:3 * u + 3]
                crs += [vy * s2 - vz * s1, vz * s0 - vx * s2,
                        vx * s1 - vy * s0]
            cr = jnp.concatenate(crs, axis=0)
        ovp = []
        for wo in range(4):
            acc = t011[wo:wo + 1] * shv
            if not first:
                for u in range(4):
                    acc = acc + w[384 + 4 * u + wo:385 + 4 * u + wo] * v_in[3 * u:3 * u + 3]
                    acc = acc + w[400 + 4 * u + wo:401 + 4 * u + wo] * cr[3 * u:3 * u + 3]
            ovp.append(acc)
        tpT = jnp.concatenate(
            [os_] + ovp + [jnp.ones((1, EB), F32), jnp.zeros((3, EB), F32)],
            axis=0)
        tpT = jnp.where(lane < N_EDGES, tpT, 0.0)
        tp[...] = lax.dot_general(tpT, jnp.eye(32, dtype=F32),
                                  (((0,), (0,)), ((), ())),
                                  preferred_element_type=F32)

    full = lambda a: pl.BlockSpec(a.shape, lambda i: (0,) * a.ndim)
    return pl.pallas_call(
        body,
        grid=(NBLK,),
        in_specs=[pl.BlockSpec((24, EB), lambda i: (0, i)),
                  pl.BlockSpec((EB, 16), lambda i: (i, 0)),
                  pl.BlockSpec((EB, gw), lambda i: (i, 0)),
                  full(w1t), full(b1c), full(w2t), full(b2c)],
        out_specs=pl.BlockSpec((EB, 32), lambda i: (i, 0)),
        out_shape=jax.ShapeDtypeStruct((EPAD, 32), F32),
    )(edgeT, gs, gd, w1t, b1c, w2t, b2c)


def _update(xf, p0, p1):
    def body(x, a, b, nf, ns_):
        s = a[...] + b[...]
        cnt = jnp.maximum(s[:, 28:29], 1.0)
        xn = x[...] + jnp.concatenate(
            [s[:, :28] / cnt, jnp.zeros((NB, 4), F32)], axis=1)
        nf[...] = xn
        ns_[...] = xn[:, :16]

    bs = pl.BlockSpec((NB, 32), lambda i: (i, 0))
    return pl.pallas_call(
        body,
        grid=(N_NODES // NB,),
        in_specs=[bs, bs, bs],
        out_specs=[bs, pl.BlockSpec((NB, 16), lambda i: (i, 0))],
        out_shape=[jax.ShapeDtypeStruct((N_NODES, 32), F32),
                   jax.ShapeDtypeStruct((N_NODES, 16), F32)],
    )(xf, p0, p1)


def _final(xf, p0, p1, wfin):
    def body(x, a, b, wf, out):
        s = a[...] + b[...]
        cnt = jnp.maximum(s[:, 28:29], 1.0)
        xn = x[...] + jnp.concatenate(
            [s[:, :28] / cnt, jnp.zeros((NB, 4), F32)], axis=1)
        out[...] = _mm(xn, wf[...])

    bs = pl.BlockSpec((NB, 32), lambda i: (i, 0))
    return pl.pallas_call(
        body,
        grid=(N_NODES // NB,),
        in_specs=[bs, bs, bs, pl.BlockSpec((32, 128), lambda i: (0, 0))],
        out_specs=pl.BlockSpec((NB, 128), lambda i: (i, 0)),
        out_shape=jax.ShapeDtypeStruct((N_NODES, 128), F32),
    )(xf, p0, p1, wfin)


def kernel(pos, node_attr, edge_attr, edge_index, params):
    p = params
    src = edge_index[0].astype(jnp.int32)
    dst = edge_index[1].astype(jnp.int32)
    padi = jnp.zeros((EPAD - N_EDGES,), jnp.int32)
    src_p = jnp.concatenate([src, padi])
    dst_p = jnp.concatenate([dst, padi])
    posp = jnp.pad(pos.astype(F32), ((0, 0), (0, 13)))
    edge_attr_p = jnp.pad(edge_attr.astype(F32), ((0, EPAD - N_EDGES), (0, 0)))
    zeros_tab = jnp.zeros((N_NODES, 32), F32)

    xf, xs = _node_encode(node_attr, p)
    psrc, pdst = _sc_gather_multi([posp, posp], [0, 1], src_p, dst_p)
    gs0, gd0 = _sc_gather_multi([xs, xs], [0, 1], src_p, dst_p)
    edgeT = _edge_pre(edge_attr_p, psrc, pdst, p)

    scal = jnp.asarray(_S416)
    out = None
    for i in range(2):
        first = i == 0
        w1t = p['fc_w1'][i].T.astype(jnp.bfloat16)
        b1c = p['fc_b1'][i].reshape(128, 1)
        w2t = (p['fc_w2'][i] * scal[None, :]).T
        b2c = (p['fc_b2'][i] * scal).reshape(416, 1)
        if first:
            w2t = jnp.concatenate([w2t[0:256], w2t[320:384]])
            b2c = jnp.concatenate([b2c[0:256], b2c[320:384]])
        w2t = w2t.astype(jnp.bfloat16)
        if first:
            gs, gd = gs0, gd0
        else:
            gs, gd = _sc_gather_multi([xs, xf], [0, 1], src_p, dst_p)
        tp = _conv(edgeT, gs, gd, w1t, b1c, w2t, b2c, first)
        parts = _sc_scatter_add(tp, src_p, zeros_tab)
        if i == 0:
            xf, xs = _update(xf, parts[0], parts[1])
        else:
            wv = jnp.kron(p['lf_w1'], jnp.eye(3, dtype=F32))
            wcat = jnp.zeros((32, 128), F32)
            wcat = wcat.at[0:16, 0:32].set(p['lf_w0'])
            wcat = wcat.at[16:28, 32:128].set(wv)
            wfin = wcat[:, jnp.asarray(_OUT_PERM)]
            out = _final(xf, parts[0], parts[1], wfin)
    return out.reshape(N_NODES, 4, 8, 4)


# speedup vs baseline: 223.9521x; 1.0071x over previous
import functools

import numpy as np
import jax
import jax.numpy as jnp
from jax import lax
from jax.experimental import pallas as pl
from jax.experimental.pallas import tpu as pltpu
from jax.experimental.pallas import tpu_sc as plsc

N_NODES = 10000
N_EDGES = 160000
NS, NV = 16, 4
R_EMB_DIM, R_EMB_MAX = 32, 10.0
OUT_CH = 32

EPAD = 163840
NW = 32
PER_W = EPAD // NW
CH = 128
EB = 2560
NBLK = EPAD // EB
NB = 2000

F32 = jnp.float32
_SQ3 = float(np.sqrt(3.0))
_A0 = 1.0 / float(np.sqrt(NS + NV))
_A1 = 1.0 / float(np.sqrt(NS + 2 * NV))

_S416 = np.ones((416,), np.float32)
_S416[0:256] = _A0
_S416[256:320] = _A0 / _SQ3
_S416[320:384] = _A1
_S416[384:400] = _A1
_S416[400:416] = _A1 / float(np.sqrt(2.0))

_IDXS = np.concatenate([np.arange(OUT_CH), np.repeat(np.arange(OUT_CH), 3)])
_ORDR = np.concatenate([np.zeros(OUT_CH, np.int64), np.tile(np.arange(1, 4), OUT_CH)])
_OUT_PERM = np.argsort(_IDXS * 4 + _ORDR, kind="stable")

_RBF_OFFS = np.linspace(0.0, R_EMB_MAX, R_EMB_DIM).astype(np.float32)
_RBF_COEFF = float(-0.5 / (_RBF_OFFS[1] - _RBF_OFFS[0]) ** 2)


def _tt(x):
    eye = jnp.eye(x.shape[1], dtype=F32)
    return lax.dot_general(eye, x, (((1,), (1,)), ((), ())),
                           preferred_element_type=F32)


def _mm(a, b):
    return jnp.dot(a, b, preferred_element_type=F32)


def _sc_mesh():
    return plsc.VectorSubcoreMesh(core_axis_name="c", subcore_axis_name="s")


def _sc_gather_multi(tabs, sels, idx0, idx1):
    ns = len(tabs)
    ws = [t.shape[1] for t in tabs]
    nch0, nch1 = 56, 24
    c0_total = 16 * nch0 * CH
    scratch = [pltpu.VMEM((nch0 * CH,), jnp.int32),
               pltpu.VMEM((nch0 * CH,), jnp.int32)]
    for w in ws:
        scratch += [pltpu.VMEM((CH, w), F32), pltpu.VMEM((CH, w), F32)]
    scratch += [pltpu.SemaphoreType.DMA] * (2 * ns)

    @functools.partial(
        pl.kernel,
        mesh=_sc_mesh(),
        compiler_params=pltpu.CompilerParams(use_tc_tiling_on_sc=False),
        out_type=tuple(jax.ShapeDtypeStruct((EPAD, w), F32) for w in ws),
        scratch_types=scratch,
    )
    def k(*args):
        tabs_r = args[:ns]
        i0_h, i1_h = args[ns], args[ns + 1]
        outs = args[ns + 2:2 * ns + 2]
        i0_v, i1_v = args[2 * ns + 2], args[2 * ns + 3]
        bufs = args[2 * ns + 4:4 * ns + 4]
        sems = args[4 * ns + 4:]
        iv = (i0_v, i1_v)
        ci = lax.axis_index("c")
        si = lax.axis_index("s")

        def gath(s, buf_i, off, sem_i):
            pltpu.async_copy(
                tabs_r[s].at[iv[sels[s]].at[pl.ds(off, CH)]],
                bufs[2 * s + buf_i], sems[2 * s + sem_i])

        def drain(s, buf_i, sem_i):
            pltpu.make_async_copy(tabs_r[s].at[pl.ds(0, CH)],
                                  bufs[2 * s + buf_i],
                                  sems[2 * s + sem_i]).wait()

        def run(nch, base):
            pltpu.sync_copy(i0_h.at[pl.ds(base, nch * CH)],
                            i0_v.at[pl.ds(0, nch * CH)])
            pltpu.sync_copy(i1_h.at[pl.ds(base, nch * CH)],
                            i1_v.at[pl.ds(0, nch * CH)])
            for s in range(ns):
                gath(s, 0, 0, 0)

            @pl.loop(0, nch, step=2)
            def _(g):
                off = g * CH
                for s in range(ns):
                    gath(s, 1, off + CH, 1)
                for s in range(ns):
                    drain(s, 0, 0)
                    pltpu.sync_copy(bufs[2 * s],
                                    outs[s].at[pl.ds(base + off, CH)])

                @pl.when(g + 2 < nch)
                def _():
                    for s in range(ns):
                        gath(s, 0, off + 2 * CH, 0)

                for s in range(ns):
                    drain(s, 1, 1)
                    pltpu.sync_copy(bufs[2 * s + 1],
                                    outs[s].at[pl.ds(base + off + CH, CH)])

        @pl.when(ci == 0)
        def _():
            run(nch0, si * nch0 * CH)

        @pl.when(ci == 1)
        def _():
            run(nch1, c0_total + si * nch1 * CH)

    return k(*tabs, idx0, idx1)


def _sc_scatter_add(tp, idx, zeros_tab):
    rps = N_NODES // 16

    @functools.partial(
        pl.kernel,
        mesh=_sc_mesh(),
        compiler_params=pltpu.CompilerParams(use_tc_tiling_on_sc=False),
        out_type=jax.ShapeDtypeStruct((2, N_NODES, 32), F32),
        scratch_types=[pltpu.VMEM((CH,), jnp.int32),
                       pltpu.VMEM((CH,), jnp.int32),
                       pltpu.VMEM((CH, 32), F32),
                       pltpu.VMEM((CH, 32), F32),
                       pltpu.VMEM_SHARED((N_NODES, 32), F32),
                       pltpu.SemaphoreType.DMA,
                       pltpu.SemaphoreType.DMA],
    )
    def k(tp_h, idx_h, z_h, out_h, ia0, ia1, r0, r1, shared, s0, s1):
        ci = lax.axis_index("c")
        si = lax.axis_index("s")
        wid = si * 2 + ci
        pltpu.sync_copy(z_h.at[pl.ds(si * rps, rps)],
                        shared.at[pl.ds(si * rps, rps)])
        plsc.subcore_barrier()
        base = wid * PER_W
        nch = PER_W // CH
        pltpu.sync_copy(idx_h.at[pl.ds(base, CH)], ia0)
        pltpu.async_copy(tp_h.at[pl.ds(base, CH)], r0, s0)

        @pl.loop(0, nch, step=2)
        def _(g):
            off = base + g * CH
            pltpu.sync_copy(idx_h.at[pl.ds(off + CH, CH)], ia1)
            c1 = pltpu.async_copy(tp_h.at[pl.ds(off + CH, CH)], r1, s1)
            pltpu.make_async_copy(tp_h.at[pl.ds(0, CH)], r0, s0).wait()
            pltpu.sync_copy(r0, shared.at[ia0], add=True)

            @pl.when(g + 2 < nch)
            def _():
                pltpu.sync_copy(idx_h.at[pl.ds(off + 2 * CH, CH)], ia0)
                pltpu.async_copy(tp_h.at[pl.ds(off + 2 * CH, CH)], r0, s0)

            c1.wait()
            pltpu.sync_copy(r1, shared.at[ia1], add=True)

        plsc.subcore_barrier()
        pltpu.sync_copy(shared.at[pl.ds(si * rps, rps)],
                        out_h.at[ci].at[pl.ds(si * rps, rps)])

    return k(tp, idx, zeros_tab)


def _node_encode(node_attr, p):
    def body(na, ew, eb, w1, b1, w2, b2, xf, xs):
        h = _mm(na[...], ew[...]) + eb[...]
        h = jnp.maximum(_mm(h, w1[...]) + b1[...], 0.0)
        h = _mm(h, w2[...]) + b2[...]
        xs[...] = h
        xf[...] = jnp.concatenate([h, jnp.zeros((NB, 16), F32)], axis=1)

    full = lambda a: pl.BlockSpec(a.shape, lambda i: (0,) * a.ndim)
    args = (node_attr, p['enc_w'], p['enc_b'].reshape(1, NS),
            p['ne_w1'], p['ne_b1'].reshape(1, NS),
            p['ne_w2'], p['ne_b2'].reshape(1, NS))
    return pl.pallas_call(
        body,
        grid=(N_NODES // NB,),
        in_specs=[pl.BlockSpec((NB, 128), lambda i: (i, 0))] + [full(a) for a in args[1:]],
        out_specs=[pl.BlockSpec((NB, 32), lambda i: (i, 0)),
                   pl.BlockSpec((NB, 16), lambda i: (i, 0))],
        out_shape=[jax.ShapeDtypeStruct((N_NODES, 32), F32),
                   jax.ShapeDtypeStruct((N_NODES, 16), F32)],
    )(*args)


def _edge_pre(edge_attr, psrc, pdst, p):
    step = float(R_EMB_MAX / (R_EMB_DIM - 1))

    def body(ea, ps, pd, ew1, eb1, ew2, eb2, rw1, rb1, rw2, rb2, outT):
        i = pl.program_id(0)
        offs = step * lax.broadcasted_iota(
            jnp.int32, (R_EMB_DIM, 1), 0).astype(F32)
        lane = lax.broadcasted_iota(jnp.int32, (1, EB), 1) + i * EB
        eT = _tt(ea[...])
        psT = _tt(ps[...])
        pdT = _tt(pd[...])
        ev = psT[0:3] - pdT[0:3]
        d = jnp.sqrt(ev[0:1] * ev[0:1] + ev[1:2] * ev[1:2]
                     + ev[2:3] * ev[2:3] + 1e-12)
        emb = jnp.exp(_RBF_COEFF * (d - offs) ** 2)
        h1 = jnp.maximum(_mm(ew1[...], eT) + eb1[...], 0.0)
        e1 = _mm(ew2[...], h1) + eb2[...]
        h2 = jnp.maximum(_mm(rw1[...], emb) + rb1[...], 0.0)
        e2 = _mm(rw2[...], h2) + rb2[...]
        inv = 1.0 / d
        shv = _SQ3 * jnp.concatenate(
            [ev[1:2] * inv, ev[2:3] * inv, ev[0:1] * inv], axis=0)
        res = jnp.concatenate(
            [e1 + e2, shv, jnp.zeros((5, EB), F32)], axis=0)
        outT[...] = jnp.where(lane < N_EDGES, res, 0.0)

    full = lambda a: pl.BlockSpec(a.shape, lambda i: (0,) * a.ndim)
    wargs = (p['ee_w1'].T, p['ee_b1'].reshape(NS, 1),
             p['ee_w2'].T, p['ee_b2'].reshape(NS, 1),
             p['re_w1'].T, p['re_b1'].reshape(NS, 1),
             p['re_w2'].T, p['re_b2'].reshape(NS, 1))
    return pl.pallas_call(
        body,
        grid=(NBLK,),
        in_specs=[pl.BlockSpec((EB, 16), lambda i: (i, 0)),
                  pl.BlockSpec((EB, 16), lambda i: (i, 0)),
                  pl.BlockSpec((EB, 16), lambda i: (i, 0))] + [full(a) for a in wargs],
        out_specs=pl.BlockSpec((24, EB), lambda i: (0, i)),
        out_shape=jax.ShapeDtypeStruct((24, EPAD), F32),
    )(edge_attr, psrc, pdst, *wargs)


def _conv(edgeT, gs, gd, w1t, b1c, w2t, b2c, first):
    gw = gd.shape[1]
    t0 = 256 if first else 320

    def body(et, gs_r, gd_r, w1, b1, w2, b2, tp):
        i = pl.program_id(0)
        lane = lax.broadcasted_iota(jnp.int32, (1, EB), 1) + i * EB
        eT = et[0:16]
        shv = et[16:19]
        gsT = _tt(gs_r[...])
        gdT = _tt(gd_r[...])
        s_in = gdT[0:16]
        eaT = jnp.concatenate([eT, gsT, s_in], axis=0)
        z = jnp.maximum(
            jnp.dot(w1[...], eaT.astype(jnp.bfloat16),
                    preferred_element_type=F32) + b1[...], 0.0)
        w = jnp.dot(w2[...], z.astype(jnp.bfloat16),
                    preferred_element_type=F32) + b2[...]

        s0, s1, s2 = shv[0:1], shv[1:2], shv[2:3]
        os_ = w[0:16] * s_in[0:1]
        for u in range(1, 16):
            os_ = os_ + w[16 * u:16 * u + 16] * s_in[u:u + 1]
        t011 = w[t0:t0 + 4] * s_in[0:1]
        for u in range(1, 16):
            t011 = t011 + w[t0 + 4 * u:t0 + 4 * u + 4] * s_in[u:u + 1]
        if not first:
            v_in = gdT[16:28]
            for u in range(4):
                dv = (v_in[3 * u:3 * u + 1] * s0
                      + v_in[3 * u + 1:3 * u + 2] * s1
                      + v_in[3 * u + 2:3 * u + 3] * s2)
                os_ = os_ + w[256 + 16 * u:256 + 16 * u + 16] * dv
            crs = []
            for u in range(4):
                vx = v_in[3 * u:3 * u + 1]
                vy = v_in[3 * u + 1:3 * u + 2]
                vz = v_in[3 * u + 2:3 * u + 3]
                crs += [vy * s2 - vz * s1, vz * s0 - vx * s2,
                        vx * s1 - vy * s0]
            cr = jnp.concatenate(crs, axis=0)
        ovp = []
        for wo in range(4):
            acc = t011[wo:wo + 1] * shv
            if not first:
                for u in range(4):
                    acc = acc + w[384 + 4 * u + wo:385 + 4 * u + wo] * v_in[3 * u:3 * u + 3]
                    acc = acc + w[400 + 4 * u + wo:401 + 4 * u + wo] * cr[3 * u:3 * u + 3]
            ovp.append(acc)
        tpT = jnp.concatenate(
            [os_] + ovp + [jnp.ones((1, EB), F32), jnp.zeros((3, EB), F32)],
            axis=0)
        tpT = jnp.where(lane < N_EDGES, tpT, 0.0)
        tp[...] = lax.dot_general(tpT, jnp.eye(32, dtype=F32),
                                  (((0,), (0,)), ((), ())),
                                  preferred_element_type=F32)

    full = lambda a: pl.BlockSpec(a.shape, lambda i: (0,) * a.ndim)
    return pl.pallas_call(
        body,
        grid=(NBLK,),
        in_specs=[pl.BlockSpec((24, EB), lambda i: (0, i)),
                  pl.BlockSpec((EB, 16), lambda i: (i, 0)),
                  pl.BlockSpec((EB, gw), lambda i: (i, 0)),
                  full(w1t), full(b1c), full(w2t), full(b2c)],
        out_specs=pl.BlockSpec((EB, 32), lambda i: (i, 0)),
        out_shape=jax.ShapeDtypeStruct((EPAD, 32), F32),
    )(edgeT, gs, gd, w1t, b1c, w2t, b2c)


def _update(xf, p0, p1):
    def body(x, a, b, nf, ns_):
        s = a[...] + b[...]
        cnt = jnp.maximum(s[:, 28:29], 1.0)
        xn = x[...] + jnp.concatenate(
            [s[:, :28] / cnt, jnp.zeros((NB, 4), F32)], axis=1)
        nf[...] = xn
        ns_[...] = xn[:, :16]

    bs = pl.BlockSpec((NB, 32), lambda i: (i, 0))
    return pl.pallas_call(
        body,
        grid=(N_NODES // NB,),
        in_specs=[bs, bs, bs],
        out_specs=[bs, pl.BlockSpec((NB, 16), lambda i: (i, 0))],
        out_shape=[jax.ShapeDtypeStruct((N_NODES, 32), F32),
                   jax.ShapeDtypeStruct((N_NODES, 16), F32)],
    )(xf, p0, p1)


def _final(xf, p0, p1, wfin):
    def body(x, a, b, wf, out):
        s = a[...] + b[...]
        cnt = jnp.maximum(s[:, 28:29], 1.0)
        xn = x[...] + jnp.concatenate(
            [s[:, :28] / cnt, jnp.zeros((NB, 4), F32)], axis=1)
        out[...] = _mm(xn, wf[...])

    bs = pl.BlockSpec((NB, 32), lambda i: (i, 0))
    return pl.pallas_call(
        body,
        grid=(N_NODES // NB,),
        in_specs=[bs, bs, bs, pl.BlockSpec((32, 128), lambda i: (0, 0))],
        out_specs=pl.BlockSpec((NB, 128), lambda i: (i, 0)),
        out_shape=jax.ShapeDtypeStruct((N_NODES, 128), F32),
    )(xf, p0, p1, wfin)


def kernel(pos, node_attr, edge_attr, edge_index, params):
    p = params
    src = edge_index[0].astype(jnp.int32)
    dst = edge_index[1].astype(jnp.int32)
    padi = jnp.zeros((EPAD - N_EDGES,), jnp.int32)
    src_p = jnp.concatenate([src, padi])
    dst_p = jnp.concatenate([dst, padi])
    posp = jnp.pad(pos.astype(F32), ((0, 0), (0, 13)))
    edge_attr_p = jnp.pad(edge_attr.astype(F32), ((0, EPAD - N_EDGES), (0, 0)))
    zeros_tab = jnp.zeros((N_NODES, 32), F32)

    xf, xs = _node_encode(node_attr, p)
    psrc, pdst = _sc_gather_multi([posp, posp], [0, 1], src_p, dst_p)
    gs0, gd0 = _sc_gather_multi([xs, xs], [0, 1], src_p, dst_p)
    edgeT = _edge_pre(edge_attr_p, psrc, pdst, p)

    scal = jnp.asarray(_S416)
    out = None
    for i in range(2):
        first = i == 0
        w1t = p['fc_w1'][i].T.astype(jnp.bfloat16)
        b1c = p['fc_b1'][i].reshape(128, 1)
        w2t = (p['fc_w2'][i] * scal[None, :]).T
        b2c = (p['fc_b2'][i] * scal).reshape(416, 1)
        if first:
            w2t = jnp.concatenate([w2t[0:256], w2t[320:384]])
            b2c = jnp.concatenate([b2c[0:256], b2c[320:384]])
        w2t = w2t.astype(jnp.bfloat16)
        if first:
            gs, gd = gs0, gd0
        else:
            gs, gd = _sc_gather_multi([xs, xf], [0, 1], src_p, dst_p)
        tp = _conv(edgeT, gs, gd, w1t, b1c, w2t, b2c, first)
        parts = _sc_scatter_add(tp, src_p, zeros_tab)
        if i == 0:
            xf, xs = _update(xf, parts[0], parts[1])
        else:
            wv = jnp.kron(p['lf_w1'], jnp.eye(3, dtype=F32))
            wcat = jnp.zeros((32, 128), F32)
            wcat = wcat.at[0:16, 0:32].set(p['lf_w0'])
            wcat = wcat.at[16:28, 32:128].set(wv)
            wfin = wcat[:, jnp.asarray(_OUT_PERM)]
            out = _final(xf, parts[0], parts[1], wfin)
    return out.reshape(N_NODES, 4, 8, 4)
